# Initial kernel scaffold; baseline (speedup 1.0000x reference)
import jax
import jax.numpy as jnp
from jax.experimental import pallas as pl


def kernel(user_ing, recipe_ing, user_table, recipe_table, W1, b1, W2, b2, W3, b3):
    raise NotImplementedError("write your pallas kernel here")


# speedup vs baseline: 1.0713x; 1.0713x over previous
import functools

import jax
import jax.numpy as jnp
from jax import lax
from jax.experimental import pallas as pl
from jax.experimental.pallas import tpu as pltpu
from jax.experimental.pallas import tpu_sc as plsc

LANES = 16
NUM_WORKERS = 32


def _pool_kernel(B, L, E, C):
    b_per_w = B // NUM_WORKERS
    n_chunks = b_per_w // C
    n_vregs = E // LANES
    inv_l = 1.0 / L

    mesh = plsc.VectorSubcoreMesh(
        core_axis_name="c", subcore_axis_name="s",
        num_cores=2, num_subcores=16)

    def accum_pool(rows_v, p, out_v, col_off):
        zero = jnp.zeros((LANES,), jnp.float32)

        @pl.loop(0, L // 5, init_carry=tuple(zero for _ in range(n_vregs)))
        def body(r5, accs):
            r = r5 * 5
            for dr in range(5):
                accs = tuple(
                    a + rows_v[p, r + dr, pl.ds(k * LANES, LANES)]
                    for k, a in enumerate(accs)
                )
            return accs

        for k in range(n_vregs):
            out_v[p, pl.ds(col_off + k * LANES, LANES)] = body[k] * inv_l

    @functools.partial(
        pl.kernel,
        out_type=jax.ShapeDtypeStruct((B, 2 * E), jnp.float32),
        mesh=mesh,
        scratch_types=[
            pltpu.VMEM((C, L), jnp.int32),
            pltpu.VMEM((C, L), jnp.int32),
            pltpu.VMEM((C, L, E), jnp.float32),
            pltpu.VMEM((C, L, E), jnp.float32),
            pltpu.VMEM((C, 2 * E), jnp.float32),
            pltpu.SemaphoreType.DMA,
        ],
        compiler_params=pltpu.CompilerParams(use_tc_tiling_on_sc=False),
    )
    def pool(user_ing, recipe_ing, user_table, recipe_table, out_hbm,
             idx_u, idx_r, rows_u, rows_r, out_v, sem):
        wid = lax.axis_index("s") * 2 + lax.axis_index("c")
        wbase = wid * b_per_w

        @pl.loop(0, n_chunks)
        def chunk_loop(c):
            base = wbase + c * C
            pltpu.sync_copy(user_ing.at[pl.ds(base, C)], idx_u)
            pltpu.sync_copy(recipe_ing.at[pl.ds(base, C)], idx_r)
            cps = []
            for p in range(C):
                cps.append(pltpu.async_copy(
                    user_table.at[idx_u.at[p]], rows_u.at[p], sem))
                cps.append(pltpu.async_copy(
                    recipe_table.at[idx_r.at[p]], rows_r.at[p], sem))
            for cp in cps:
                cp.wait()
            for p in range(C):
                accum_pool(rows_u, p, out_v, 0)
                accum_pool(rows_r, p, out_v, E)
            pltpu.sync_copy(out_v, out_hbm.at[pl.ds(base, C)])

    return pool


def _mlp_kernel(B, E, H1, H2, BLK):
    def body(x_ref, w1_ref, b1_ref, w2_ref, b2_ref, w3_ref, b3_ref, o_ref):
        x = x_ref[...]
        h = jnp.dot(x, w1_ref[...], preferred_element_type=jnp.float32)
        h = jnp.maximum(h + b1_ref[...], 0.0)
        h = jnp.dot(h, w2_ref[...], preferred_element_type=jnp.float32)
        h = jnp.maximum(h + b2_ref[...], 0.0)
        o = jnp.sum(h * w3_ref[...], axis=1, keepdims=True)
        o_ref[...] = o + b3_ref[...]

    grid = (B // BLK,)
    return pl.pallas_call(
        body,
        grid=grid,
        in_specs=[
            pl.BlockSpec((BLK, 2 * E), lambda i: (i, 0)),
            pl.BlockSpec((2 * E, H1), lambda i: (0, 0)),
            pl.BlockSpec((1, H1), lambda i: (0, 0)),
            pl.BlockSpec((H1, H2), lambda i: (0, 0)),
            pl.BlockSpec((1, H2), lambda i: (0, 0)),
            pl.BlockSpec((1, H2), lambda i: (0, 0)),
            pl.BlockSpec((1, 1), lambda i: (0, 0)),
        ],
        out_specs=pl.BlockSpec((BLK, 1), lambda i: (i, 0)),
        out_shape=jax.ShapeDtypeStruct((B, 1), jnp.float32),
    )


def kernel(user_ing, recipe_ing, user_table, recipe_table, W1, b1, W2, b2, W3, b3):
    B, L = user_ing.shape
    V, E = user_table.shape
    H1 = W1.shape[0]
    H2 = W2.shape[0]

    pooled = _pool_kernel(B, L, E, C=8)(
        user_ing, recipe_ing, user_table, recipe_table)

    out = _mlp_kernel(B, E, H1, H2, BLK=1024)(
        pooled,
        W1.T, b1.reshape(1, H1),
        W2.T, b2.reshape(1, H2),
        W3.reshape(1, H2), b3.reshape(1, 1),
    )
    return out[:, 0]
